# baseline (speedup 1.0000x reference)
import jax
import jax.numpy as jnp
from jax.experimental import pallas as pl
from jax.experimental.pallas import tpu as pltpu


def kernel(x, person_emb):
    B, T, P, D = x.shape

    def body(emb_ref, o_ref, buf, sem):
        buf[...] = jnp.broadcast_to(emb_ref[...][None, None], (1, T, P, D))
        pltpu.make_async_copy(buf, o_ref.at[pl.ds(0, 1)], sem).start()
        pltpu.make_async_copy(buf, o_ref.at[pl.ds(0, 1)], sem).wait()
        have = 1
        while have < B:
            n = min(have, B - have)
            cp = pltpu.make_async_copy(
                o_ref.at[pl.ds(0, n)], o_ref.at[pl.ds(have, n)], sem
            )
            cp.start()
            cp.wait()
            have += n

    return pl.pallas_call(
        body,
        in_specs=[pl.BlockSpec(memory_space=pltpu.VMEM)],
        out_specs=pl.BlockSpec(memory_space=pl.ANY),
        out_shape=jax.ShapeDtypeStruct((B, T, P, D), person_emb.dtype),
        scratch_shapes=[
            pltpu.VMEM((1, T, P, D), person_emb.dtype),
            pltpu.SemaphoreType.DMA,
        ],
    )(person_emb)
